# Initial kernel scaffold; baseline (speedup 1.0000x reference)
import jax
import jax.numpy as jnp
from jax.experimental import pallas as pl


def kernel(char_ids, offsets, prev_tag_ids, word_ids, char_table, tag_table, word_table, W_w, W_b):
    raise NotImplementedError("write your pallas kernel here")


# speedup vs baseline: 1.0567x; 1.0567x over previous
import functools

import jax
import jax.numpy as jnp
from jax import lax
from jax.experimental import pallas as pl
from jax.experimental.pallas import tpu as pltpu
from jax.experimental.pallas import tpu_sc as plsc

B = 4096
H = 256
TAGS = 19
N_BAGS = 3 * B
TOTAL = 73728
CH = 64
NW = 32
BAGS_PER_W = N_BAGS // NW
TRASH = BAGS_PER_W


def _sc_embed(cid_hbm, seg_hbm, off_hbm, wf_hbm, tag_ids_hbm,
              ctab_hbm, ttab_hbm, wtab_hbm, x_hbm,
              idsbuf, segbuf, rows, offbuf, acc, gsem):
    c = lax.axis_index("c")
    s = lax.axis_index("s")
    w = c * 16 + s
    bag0 = w * BAGS_PER_W
    lanes = lax.iota(jnp.int32, 16)

    for k in range(BAGS_PER_W // CH):
        pltpu.sync_copy(wf_hbm.at[pl.ds(bag0 + k * CH, CH)], idsbuf)
        pltpu.async_copy(wtab_hbm.at[idsbuf],
                         acc.at[pl.ds(k * CH, CH)], gsem).wait()

    for k in range(BAGS_PER_W // 3 // CH):
        pltpu.sync_copy(tag_ids_hbm.at[pl.ds(w * (BAGS_PER_W // 3) + k * CH, CH)],
                        idsbuf)
        pltpu.async_copy(ttab_hbm.at[idsbuf], rows, gsem).wait()

        def tag_add(t, carry):
            r = 3 * (k * CH + t)
            for q in range(H // 16):
                sl = pl.ds(q * 16, 16)
                acc[r, sl] = acc[r, sl] + rows[t, sl]
            return carry

        lax.fori_loop(0, CH, tag_add, 0)

    pltpu.sync_copy(off_hbm.at[pl.ds(bag0, 16)], offbuf)
    p_start = offbuf[...][0]
    pltpu.sync_copy(off_hbm.at[pl.ds(bag0 + BAGS_PER_W, 16)], offbuf)
    p_end = offbuf[...][0]
    p0 = (p_start // 8) * 8
    nch = (p_end - p0 + CH - 1) // CH

    def chunk(kk, carry):
        base = p0 + kk * CH
        pltpu.sync_copy(cid_hbm.at[pl.ds(base, CH)], idsbuf)
        pltpu.sync_copy(seg_hbm.at[pl.ds(base, CH)], segbuf)
        pltpu.async_copy(ctab_hbm.at[idsbuf], rows, gsem).wait()

        def vreg16(j, carry2):
            pos = base + j * 16 + lanes
            valid = (pos >= p_start) & (pos < p_end)
            sv = jnp.where(valid, segbuf[pl.ds(j * 16, 16)] - bag0, TRASH)
            for l in range(16):
                r = sv[l]
                for q in range(H // 16):
                    sl = pl.ds(q * 16, 16)
                    acc[r, sl] = acc[r, sl] + rows[j * 16 + l, sl]
            return carry2

        lax.fori_loop(0, CH // 16, vreg16, 0)
        return carry

    lax.fori_loop(0, nch, chunk, 0)

    pltpu.sync_copy(acc.at[pl.ds(0, BAGS_PER_W)],
                    x_hbm.at[pl.ds(bag0, BAGS_PER_W)])


_sc_embed_call = functools.partial(
    pl.kernel,
    out_type=jax.ShapeDtypeStruct((N_BAGS, H), jnp.float32),
    mesh=plsc.VectorSubcoreMesh(core_axis_name="c", subcore_axis_name="s"),
    scratch_types=[
        pltpu.VMEM((CH,), jnp.int32),
        pltpu.VMEM((CH,), jnp.int32),
        pltpu.VMEM((CH, H), jnp.float32),
        pltpu.VMEM((16,), jnp.int32),
        pltpu.VMEM((BAGS_PER_W + 1, H), jnp.float32),
        pltpu.SemaphoreType.DMA,
    ],
)(_sc_embed)


def _tc_classifier(x_ref, w_ref, b_ref, o_ref):
    y = jnp.dot(x_ref[...], w_ref[...], preferred_element_type=jnp.float32)
    y = jnp.maximum(y + b_ref[...], 0.0)
    y = y - jnp.max(y, axis=1, keepdims=True)
    o_ref[...] = jnp.exp(y)


def kernel(char_ids, offsets, prev_tag_ids, word_ids,
           char_table, tag_table, word_table, W_w, W_b):
    offsets = offsets.astype(jnp.int32)
    seg = (jnp.searchsorted(offsets, jnp.arange(TOTAL, dtype=jnp.int32),
                            side="right").astype(jnp.int32) - 1)
    cid_pad = jnp.concatenate(
        [char_ids.astype(jnp.int32), jnp.zeros((CH,), jnp.int32)])
    seg_pad = jnp.concatenate([seg, jnp.zeros((CH,), jnp.int32)])
    off_pad = jnp.concatenate([offsets, jnp.full((16,), TOTAL, jnp.int32)])
    wf = word_ids.reshape(-1).astype(jnp.int32)

    x = _sc_embed_call(cid_pad, seg_pad, off_pad, wf,
                       prev_tag_ids.astype(jnp.int32),
                       char_table, tag_table, word_table)
    x = x.reshape(B, 3 * H)

    blk = 512
    out = pl.pallas_call(
        _tc_classifier,
        grid=(B // blk,),
        in_specs=[
            pl.BlockSpec((blk, 3 * H), lambda i: (i, 0)),
            pl.BlockSpec((3 * H, TAGS), lambda i: (0, 0)),
            pl.BlockSpec((1, TAGS), lambda i: (0, 0)),
        ],
        out_specs=pl.BlockSpec((blk, TAGS), lambda i: (i, 0)),
        out_shape=jax.ShapeDtypeStruct((B, TAGS), jnp.float32),
    )(x, W_w.T, W_b.reshape(1, TAGS))
    return out


# speedup vs baseline: 24.9768x; 23.6376x over previous
import functools

import jax
import jax.numpy as jnp
from jax import lax
from jax.experimental import pallas as pl
from jax.experimental.pallas import tpu as pltpu
from jax.experimental.pallas import tpu_sc as plsc

B = 4096
H = 256
TAGS = 19
N_BAGS = 3 * B
TOTAL = 73728
CH = 64
NW = 32
BAGS_PER_W = N_BAGS // NW
TRASH = BAGS_PER_W
NOFF = BAGS_PER_W // 16


def _sc_embed(cid_hbm, off_hbm, wf_hbm, tag_ids_hbm,
              ctab_hbm, ttab_hbm, wtab_hbm, x_hbm,
              idsbuf, rows, offbuf, mbuf, acc, gsem):
    c = lax.axis_index("c")
    s = lax.axis_index("s")
    w = c * 16 + s
    bag0 = w * BAGS_PER_W
    lanes = lax.iota(jnp.int32, 16)

    for k in range(BAGS_PER_W // CH):
        pltpu.sync_copy(wf_hbm.at[pl.ds(bag0 + k * CH, CH)], idsbuf)
        pltpu.async_copy(wtab_hbm.at[idsbuf],
                         acc.at[pl.ds(k * CH, CH)], gsem).wait()

    for k in range(BAGS_PER_W // 3 // CH):
        pltpu.sync_copy(tag_ids_hbm.at[pl.ds(w * (BAGS_PER_W // 3) + k * CH, CH)],
                        idsbuf)
        pltpu.async_copy(ttab_hbm.at[idsbuf], rows, gsem).wait()

        def tag_add(t, carry):
            r = 3 * (k * CH + t)
            for q in range(H // 16):
                sl = pl.ds(q * 16, 16)
                acc[r, sl] = acc[r, sl] + rows[t, sl]
            return carry

        lax.fori_loop(0, CH, tag_add, 0)

    pltpu.sync_copy(off_hbm.at[pl.ds(bag0, 400)], offbuf)
    p_start = offbuf[pl.ds(0, 16)][0]
    p_end = offbuf[pl.ds(BAGS_PER_W, 16)][0]
    p0 = (p_start // 8) * 8
    nch = (p_end - p0 + CH - 1) // CH

    def chunk(kk, carry):
        base = p0 + kk * CH
        pltpu.sync_copy(cid_hbm.at[pl.ds(base, CH)], idsbuf)
        gdesc = pltpu.async_copy(ctab_hbm.at[idsbuf], rows, gsem)

        for j in range(CH // 16):
            mbuf[pl.ds(j * 16, 16)] = jnp.zeros((16,), jnp.int32)
        for j in range(NOFF):
            offv = offbuf[pl.ds(j * 16, 16)]
            offn = offbuf[pl.ds(j * 16 + 1, 16)]
            m = (offv >= base) & (offv < base + CH) & (offv < offn)
            plsc.store_scatter(mbuf, [offv - base], 16 * j + lanes, mask=m)

        gdesc.wait()

        def inner(j, cr):
            pos = base + j * 16 + lanes
            mv = jnp.maximum(plsc.cummax(mbuf[pl.ds(j * 16, 16)]), cr)
            newcr = mv[15]
            valid = (pos >= p_start) & (pos < p_end)
            sv = jnp.where(valid, mv, TRASH)
            for l in range(16):
                r = sv[l]
                for q in range(H // 16):
                    sl = pl.ds(q * 16, 16)
                    acc[r, sl] = acc[r, sl] + rows[j * 16 + l, sl]
            return newcr

        return lax.fori_loop(0, CH // 16, inner, carry)

    lax.fori_loop(0, nch, chunk, 0)

    pltpu.sync_copy(acc.at[pl.ds(0, BAGS_PER_W)],
                    x_hbm.at[pl.ds(bag0, BAGS_PER_W)])


_sc_embed_call = functools.partial(
    pl.kernel,
    out_type=jax.ShapeDtypeStruct((N_BAGS, H), jnp.float32),
    mesh=plsc.VectorSubcoreMesh(core_axis_name="c", subcore_axis_name="s"),
    compiler_params=pltpu.CompilerParams(needs_layout_passes=False),
    scratch_types=[
        pltpu.VMEM((CH,), jnp.int32),
        pltpu.VMEM((CH, H), jnp.float32),
        pltpu.VMEM((400,), jnp.int32),
        pltpu.VMEM((CH,), jnp.int32),
        pltpu.VMEM((BAGS_PER_W + 1, H), jnp.float32),
        pltpu.SemaphoreType.DMA,
    ],
)(_sc_embed)


def _tc_classifier(x_ref, w_ref, b_ref, o_ref):
    y = jnp.dot(x_ref[...], w_ref[...], preferred_element_type=jnp.float32)
    y = jnp.maximum(y + b_ref[...], 0.0)
    y = y - jnp.max(y, axis=1, keepdims=True)
    o_ref[...] = jnp.exp(y)


def kernel(char_ids, offsets, prev_tag_ids, word_ids,
           char_table, tag_table, word_table, W_w, W_b):
    cid_pad = jnp.concatenate(
        [char_ids.astype(jnp.int32), jnp.zeros((CH,), jnp.int32)])
    off_pad = jnp.concatenate(
        [offsets.astype(jnp.int32), jnp.full((16,), TOTAL, jnp.int32)])
    wf = word_ids.reshape(-1).astype(jnp.int32)

    x = _sc_embed_call(cid_pad, off_pad, wf,
                       prev_tag_ids.astype(jnp.int32),
                       char_table, tag_table, word_table)
    x = x.reshape(B, 3 * H)

    blk = 512
    out = pl.pallas_call(
        _tc_classifier,
        grid=(B // blk,),
        in_specs=[
            pl.BlockSpec((blk, 3 * H), lambda i: (i, 0)),
            pl.BlockSpec((3 * H, TAGS), lambda i: (0, 0)),
            pl.BlockSpec((1, TAGS), lambda i: (0, 0)),
        ],
        out_specs=pl.BlockSpec((blk, TAGS), lambda i: (i, 0)),
        out_shape=jax.ShapeDtypeStruct((B, TAGS), jnp.float32),
    )(x, W_w.T, W_b.reshape(1, TAGS))
    return out


# speedup vs baseline: 27.0964x; 1.0849x over previous
import functools

import jax
import jax.numpy as jnp
from jax import lax
from jax.experimental import pallas as pl
from jax.experimental.pallas import tpu as pltpu
from jax.experimental.pallas import tpu_sc as plsc

B = 4096
H = 256
TAGS = 19
N_BAGS = 3 * B
TOTAL = 73728
CH = 48
NW = 32
BAGS_PER_W = N_BAGS // NW
TRASH = BAGS_PER_W
NOFF = BAGS_PER_W // 16


def _sc_embed(cid_hbm, off_hbm, wf_hbm, tag_ids_hbm,
              ctab_hbm, ttab_hbm, wtab_hbm, x_hbm,
              idsbuf, idsbuf2, tbuf, rows, rowsb, offbuf, mbuf, acc, gsem, gsemb):
    c = lax.axis_index("c")
    s = lax.axis_index("s")
    w = c * 16 + s
    bag0 = w * BAGS_PER_W
    lanes = lax.iota(jnp.int32, 16)

    for k in range(BAGS_PER_W // CH):
        pltpu.sync_copy(wf_hbm.at[pl.ds(bag0 + k * CH, CH)], idsbuf)
        pltpu.async_copy(wtab_hbm.at[idsbuf],
                         acc.at[pl.ds(k * CH, CH)], gsem).wait()

    for k in range(BAGS_PER_W // 3 // 32):
        pltpu.sync_copy(tag_ids_hbm.at[pl.ds(w * (BAGS_PER_W // 3) + k * 32, 32)],
                        tbuf)
        pltpu.async_copy(ttab_hbm.at[tbuf], rows.at[pl.ds(0, 32)], gsem).wait()

        def tag_add(t, carry):
            r = 3 * (k * 32 + t)
            for q in range(H // 16):
                sl = pl.ds(q * 16, 16)
                acc[r, sl] = acc[r, sl] + rows[t, sl]
            return carry

        lax.fori_loop(0, 32, tag_add, 0)

    pltpu.sync_copy(off_hbm.at[pl.ds(bag0, 400)], offbuf)
    p_start = offbuf[pl.ds(0, 16)][0]
    p_end = offbuf[pl.ds(BAGS_PER_W, 16)][0]
    p0 = (p_start // 8) * 8
    nch = (p_end - p0 + CH - 1) // CH

    ids2 = [idsbuf, idsbuf2]
    rows2 = [rows, rowsb]
    sem2 = [gsem, gsemb]

    def issue(kk, par):
        pltpu.sync_copy(cid_hbm.at[pl.ds(p0 + kk * CH, CH)], ids2[par])
        pltpu.async_copy(ctab_hbm.at[ids2[par]], rows2[par], sem2[par])

    @pl.when(nch > 0)
    def _():
        issue(0, 0)

    def chunk(kk, carry):
        base = p0 + kk * CH

        for j in range(CH // 16):
            mbuf[pl.ds(j * 16, 16)] = jnp.zeros((16,), jnp.int32)
        for j in range(NOFF):
            offv = offbuf[pl.ds(j * 16, 16)]
            offn = offbuf[pl.ds(j * 16 + 1, 16)]
            m = (offv >= base) & (offv < base + CH) & (offv < offn)
            plsc.store_scatter(mbuf, [offv - base], 16 * j + lanes, mask=m)

        def segfin(j, cr):
            pos = base + j * 16 + lanes
            mv = jnp.maximum(plsc.cummax(mbuf[pl.ds(j * 16, 16)]), cr)
            newcr = mv[15]
            valid = (pos >= p_start) & (pos < p_end)
            mbuf[pl.ds(j * 16, 16)] = jnp.where(valid, mv, TRASH)
            return newcr

        carry = lax.fori_loop(0, CH // 16, segfin, carry)

        par = lax.rem(kk, 2)

        @pl.when(kk + 1 < nch)
        def _():
            @pl.when(par == 0)
            def _():
                issue(kk + 1, 1)

            @pl.when(par == 1)
            def _():
                issue(kk + 1, 0)

        def rmw(p):
            pltpu.make_async_copy(
                ctab_hbm.at[pl.ds(0, CH)], rows2[p], sem2[p]).wait()

            def inner(j, cr2):
                sv = mbuf[pl.ds(j * 16, 16)]
                for l in range(16):
                    r = sv[l]
                    for q in range(H // 16):
                        sl = pl.ds(q * 16, 16)
                        acc[r, sl] = acc[r, sl] + rows2[p][j * 16 + l, sl]
                return cr2

            lax.fori_loop(0, CH // 16, inner, 0)

        @pl.when(par == 0)
        def _():
            rmw(0)

        @pl.when(par == 1)
        def _():
            rmw(1)

        return carry

    lax.fori_loop(0, nch, chunk, 0)

    pltpu.sync_copy(acc.at[pl.ds(0, BAGS_PER_W)],
                    x_hbm.at[pl.ds(bag0, BAGS_PER_W)])


_sc_embed_call = functools.partial(
    pl.kernel,
    out_type=jax.ShapeDtypeStruct((N_BAGS, H), jnp.float32),
    mesh=plsc.VectorSubcoreMesh(core_axis_name="c", subcore_axis_name="s"),
    compiler_params=pltpu.CompilerParams(needs_layout_passes=False),
    scratch_types=[
        pltpu.VMEM((CH,), jnp.int32),
        pltpu.VMEM((CH,), jnp.int32),
        pltpu.VMEM((32,), jnp.int32),
        pltpu.VMEM((CH, H), jnp.float32),
        pltpu.VMEM((CH, H), jnp.float32),
        pltpu.VMEM((400,), jnp.int32),
        pltpu.VMEM((CH,), jnp.int32),
        pltpu.VMEM((BAGS_PER_W + 1, H), jnp.float32),
        pltpu.SemaphoreType.DMA,
        pltpu.SemaphoreType.DMA,
    ],
)(_sc_embed)


def _tc_classifier(x_ref, w_ref, b_ref, o_ref):
    y = jnp.dot(x_ref[...], w_ref[...], preferred_element_type=jnp.float32)
    y = jnp.maximum(y + b_ref[...], 0.0)
    y = y - jnp.max(y, axis=1, keepdims=True)
    o_ref[...] = jnp.exp(y)


def kernel(char_ids, offsets, prev_tag_ids, word_ids,
           char_table, tag_table, word_table, W_w, W_b):
    cid_pad = jnp.concatenate(
        [char_ids.astype(jnp.int32), jnp.zeros((CH,), jnp.int32)])
    off_pad = jnp.concatenate(
        [offsets.astype(jnp.int32), jnp.full((16,), TOTAL, jnp.int32)])
    wf = word_ids.reshape(-1).astype(jnp.int32)

    x = _sc_embed_call(cid_pad, off_pad, wf,
                       prev_tag_ids.astype(jnp.int32),
                       char_table, tag_table, word_table)
    x = x.reshape(B, 3 * H)

    blk = 512
    out = pl.pallas_call(
        _tc_classifier,
        grid=(B // blk,),
        in_specs=[
            pl.BlockSpec((blk, 3 * H), lambda i: (i, 0)),
            pl.BlockSpec((3 * H, TAGS), lambda i: (0, 0)),
            pl.BlockSpec((1, TAGS), lambda i: (0, 0)),
        ],
        out_specs=pl.BlockSpec((blk, TAGS), lambda i: (i, 0)),
        out_shape=jax.ShapeDtypeStruct((B, TAGS), jnp.float32),
    )(x, W_w.T, W_b.reshape(1, TAGS))
    return out


# speedup vs baseline: 43.5750x; 1.6081x over previous
import functools

import jax
import jax.numpy as jnp
from jax import lax
from jax.experimental import pallas as pl
from jax.experimental.pallas import tpu as pltpu
from jax.experimental.pallas import tpu_sc as plsc

B = 4096
H = 256
TAGS = 19
N_BAGS = 3 * B
TOTAL = 73728
CH = 48
NW = 32
BAGS_PER_W = N_BAGS // NW
TRASH = BAGS_PER_W
NOFF = BAGS_PER_W // 16


def _sc_embed(cid_hbm, off_hbm, wf_hbm, tag_ids_hbm,
              ctab_hbm, ttab_hbm, wtab_hbm, x_hbm,
              idsbuf, idsbuf2, tbuf, rows, rowsb, offbuf, mbuf, acc, gsem, gsemb):
    c = lax.axis_index("c")
    s = lax.axis_index("s")
    w = c * 16 + s
    bag0 = w * BAGS_PER_W
    lanes = lax.iota(jnp.int32, 16)

    for k in range(BAGS_PER_W // CH):
        pltpu.sync_copy(wf_hbm.at[pl.ds(bag0 + k * CH, CH)], idsbuf)
        pltpu.async_copy(wtab_hbm.at[idsbuf],
                         acc.at[pl.ds(k * CH, CH)], gsem).wait()

    for k in range(BAGS_PER_W // 3 // 32):
        pltpu.sync_copy(tag_ids_hbm.at[pl.ds(w * (BAGS_PER_W // 3) + k * 32, 32)],
                        tbuf)
        pltpu.async_copy(ttab_hbm.at[tbuf], rows.at[pl.ds(0, 32)], gsem).wait()

        def tag_add(t, carry):
            r = 3 * (k * 32 + t)
            for q in range(H // 16):
                sl = pl.ds(q * 16, 16)
                acc[r, sl] = acc[r, sl] + rows[t, sl]
            return carry

        lax.fori_loop(0, 32, tag_add, 0)

    pltpu.sync_copy(off_hbm.at[pl.ds(bag0, 400)], offbuf)
    p_start = offbuf[pl.ds(0, 16)][0]
    p_end = offbuf[pl.ds(BAGS_PER_W, 16)][0]
    p0 = (p_start // 8) * 8
    nch = (p_end - p0 + CH - 1) // CH

    ids2 = [idsbuf, idsbuf2]
    rows2 = [rows, rowsb]
    sem2 = [gsem, gsemb]

    def issue(kk, par):
        pltpu.sync_copy(cid_hbm.at[pl.ds(p0 + kk * CH, CH)], ids2[par])
        pltpu.async_copy(ctab_hbm.at[ids2[par]], rows2[par], sem2[par])

    @pl.when(nch > 0)
    def _():
        issue(0, 0)

    NQ = H // 16
    init = ((jnp.int32(0), jnp.int32(TRASH))
            + tuple(jnp.zeros((16,), jnp.float32) for _ in range(NQ)))

    def chunk(kk, carry):
        base = p0 + kk * CH

        for j in range(CH // 16):
            mbuf[pl.ds(j * 16, 16)] = jnp.zeros((16,), jnp.int32)
        for j in range(NOFF):
            offv = offbuf[pl.ds(j * 16, 16)]
            offn = offbuf[pl.ds(j * 16 + 1, 16)]
            m = (offv >= base) & (offv < base + CH) & (offv < offn)
            plsc.store_scatter(mbuf, [offv - base], 16 * j + lanes, mask=m)

        def segfin(j, cr):
            pos = base + j * 16 + lanes
            mv = jnp.maximum(plsc.cummax(mbuf[pl.ds(j * 16, 16)]), cr)
            newcr = mv[15]
            valid = (pos >= p_start) & (pos < p_end)
            mbuf[pl.ds(j * 16, 16)] = jnp.where(valid, mv, TRASH)
            return newcr

        cr = lax.fori_loop(0, CH // 16, segfin, carry[0])

        par = lax.rem(kk, 2)

        @pl.when(kk + 1 < nch)
        def _():
            @pl.when(par == 0)
            def _():
                issue(kk + 1, 1)

            @pl.when(par == 1)
            def _():
                issue(kk + 1, 0)

        def rmw(p, st):
            pltpu.make_async_copy(
                ctab_hbm.at[pl.ds(0, CH)], rows2[p], sem2[p]).wait()

            def inner(j, st2):
                sv = mbuf[pl.ds(j * 16, 16)]
                for l in range(16):
                    sg = sv[l]
                    t = j * 16 + l
                    rowv = tuple(rows2[p][t, pl.ds(q * 16, 16)]
                                 for q in range(NQ))

                    def flush(ops):
                        cur_ = ops[0]
                        for q in range(NQ):
                            sl = pl.ds(q * 16, 16)
                            acc[cur_, sl] = acc[cur_, sl] + ops[1 + q]
                        return (sg,) + ops[1 + NQ:]

                    def cont(ops):
                        return ((ops[0],)
                                + tuple(a + r for a, r in
                                        zip(ops[1:1 + NQ], ops[1 + NQ:])))

                    st2 = lax.cond(sg != st2[0], flush, cont, st2 + rowv)
                return st2

            return lax.fori_loop(0, CH // 16, inner, st)

        st = lax.cond(par == 0,
                      lambda s: rmw(0, s),
                      lambda s: rmw(1, s),
                      carry[1:])
        return (cr,) + st

    fin = lax.fori_loop(0, nch, chunk, init)
    cur = fin[1]
    for q in range(NQ):
        sl = pl.ds(q * 16, 16)
        acc[cur, sl] = acc[cur, sl] + fin[2 + q]

    pltpu.sync_copy(acc.at[pl.ds(0, BAGS_PER_W)],
                    x_hbm.at[pl.ds(bag0, BAGS_PER_W)])


_sc_embed_call = functools.partial(
    pl.kernel,
    out_type=jax.ShapeDtypeStruct((N_BAGS, H), jnp.float32),
    mesh=plsc.VectorSubcoreMesh(core_axis_name="c", subcore_axis_name="s"),
    compiler_params=pltpu.CompilerParams(needs_layout_passes=False),
    scratch_types=[
        pltpu.VMEM((CH,), jnp.int32),
        pltpu.VMEM((CH,), jnp.int32),
        pltpu.VMEM((32,), jnp.int32),
        pltpu.VMEM((CH, H), jnp.float32),
        pltpu.VMEM((CH, H), jnp.float32),
        pltpu.VMEM((400,), jnp.int32),
        pltpu.VMEM((CH,), jnp.int32),
        pltpu.VMEM((BAGS_PER_W + 1, H), jnp.float32),
        pltpu.SemaphoreType.DMA,
        pltpu.SemaphoreType.DMA,
    ],
)(_sc_embed)


def _tc_classifier(x_ref, w_ref, b_ref, o_ref):
    y = jnp.dot(x_ref[...], w_ref[...], preferred_element_type=jnp.float32)
    y = jnp.maximum(y + b_ref[...], 0.0)
    y = y - jnp.max(y, axis=1, keepdims=True)
    o_ref[...] = jnp.exp(y)


def kernel(char_ids, offsets, prev_tag_ids, word_ids,
           char_table, tag_table, word_table, W_w, W_b):
    cid_pad = jnp.concatenate(
        [char_ids.astype(jnp.int32), jnp.zeros((CH,), jnp.int32)])
    off_pad = jnp.concatenate(
        [offsets.astype(jnp.int32), jnp.full((16,), TOTAL, jnp.int32)])
    wf = word_ids.reshape(-1).astype(jnp.int32)

    x = _sc_embed_call(cid_pad, off_pad, wf,
                       prev_tag_ids.astype(jnp.int32),
                       char_table, tag_table, word_table)
    x = x.reshape(B, 3 * H)

    blk = 512
    out = pl.pallas_call(
        _tc_classifier,
        grid=(B // blk,),
        in_specs=[
            pl.BlockSpec((blk, 3 * H), lambda i: (i, 0)),
            pl.BlockSpec((3 * H, TAGS), lambda i: (0, 0)),
            pl.BlockSpec((1, TAGS), lambda i: (0, 0)),
        ],
        out_specs=pl.BlockSpec((blk, TAGS), lambda i: (i, 0)),
        out_shape=jax.ShapeDtypeStruct((B, TAGS), jnp.float32),
    )(x, W_w.T, W_b.reshape(1, TAGS))
    return out


# speedup vs baseline: 43.5894x; 1.0003x over previous
import functools

import jax
import jax.numpy as jnp
from jax import lax
from jax.experimental import pallas as pl
from jax.experimental.pallas import tpu as pltpu
from jax.experimental.pallas import tpu_sc as plsc

B = 4096
H = 256
TAGS = 19
N_BAGS = 3 * B
TOTAL = 73728
CH = 48
NW = 32
BAGS_PER_W = N_BAGS // NW
TRASH = BAGS_PER_W
NOFF = BAGS_PER_W // 16


def _sc_embed(cid_hbm, off_hbm, wf_hbm, tag_ids_hbm,
              ctab_hbm, ttab_hbm, wtab_hbm, x_hbm,
              idsbuf, idsbuf2, tbuf, rows, rowsb, offbuf, mbuf, acc, gsem, gsemb):
    c = lax.axis_index("c")
    s = lax.axis_index("s")
    w = c * 16 + s
    bag0 = w * BAGS_PER_W
    lanes = lax.iota(jnp.int32, 16)

    for k in range(BAGS_PER_W // CH):
        pltpu.sync_copy(wf_hbm.at[pl.ds(bag0 + k * CH, CH)], idsbuf)
        pltpu.async_copy(wtab_hbm.at[idsbuf],
                         acc.at[pl.ds(k * CH, CH)], gsem).wait()

    for k in range(BAGS_PER_W // 3 // 32):
        pltpu.sync_copy(tag_ids_hbm.at[pl.ds(w * (BAGS_PER_W // 3) + k * 32, 32)],
                        tbuf)
        pltpu.async_copy(ttab_hbm.at[tbuf], rows.at[pl.ds(0, 32)], gsem).wait()

        def tag_add(t, carry):
            r = 3 * (k * 32 + t)
            for q in range(H // 16):
                sl = pl.ds(q * 16, 16)
                acc[r, sl] = acc[r, sl] + rows[t, sl]
            return carry

        lax.fori_loop(0, 32, tag_add, 0)

    pltpu.sync_copy(off_hbm.at[pl.ds(bag0, 400)], offbuf)
    p_start = offbuf[pl.ds(0, 16)][0]
    p_end = offbuf[pl.ds(BAGS_PER_W, 16)][0]
    p0 = (p_start // 8) * 8
    nch = (p_end - p0 + CH - 1) // CH

    ids2 = [idsbuf, idsbuf2]
    rows2 = [rows, rowsb]
    sem2 = [gsem, gsemb]

    voff = lax.rem(w, 8) * 500

    def issue(kk, par):
        pltpu.sync_copy(cid_hbm.at[pl.ds(p0 + kk * CH, CH)], ids2[par])
        for v in range(CH // 16):
            sl = pl.ds(v * 16, 16)
            ids2[par][sl] = ids2[par][sl] + voff
        pltpu.async_copy(ctab_hbm.at[ids2[par]], rows2[par], sem2[par])

    @pl.when(nch > 0)
    def _():
        issue(0, 0)

    NQ = H // 16
    init = ((jnp.int32(0), jnp.int32(TRASH))
            + tuple(jnp.zeros((16,), jnp.float32) for _ in range(NQ)))

    def chunk(kk, carry):
        base = p0 + kk * CH

        for j in range(CH // 16):
            mbuf[pl.ds(j * 16, 16)] = jnp.zeros((16,), jnp.int32)
        for j in range(NOFF):
            offv = offbuf[pl.ds(j * 16, 16)]
            offn = offbuf[pl.ds(j * 16 + 1, 16)]
            m = (offv >= base) & (offv < base + CH) & (offv < offn)
            plsc.store_scatter(mbuf, [offv - base], 16 * j + lanes, mask=m)

        def segfin(j, cr):
            pos = base + j * 16 + lanes
            mv = jnp.maximum(plsc.cummax(mbuf[pl.ds(j * 16, 16)]), cr)
            newcr = mv[15]
            valid = (pos >= p_start) & (pos < p_end)
            mbuf[pl.ds(j * 16, 16)] = jnp.where(valid, mv, TRASH)
            return newcr

        cr = lax.fori_loop(0, CH // 16, segfin, carry[0])

        par = lax.rem(kk, 2)

        @pl.when(kk + 1 < nch)
        def _():
            @pl.when(par == 0)
            def _():
                issue(kk + 1, 1)

            @pl.when(par == 1)
            def _():
                issue(kk + 1, 0)

        def rmw(p, st):
            pltpu.make_async_copy(
                ctab_hbm.at[pl.ds(0, CH)], rows2[p], sem2[p]).wait()

            def inner(j, st2):
                sv = mbuf[pl.ds(j * 16, 16)]
                for l in range(16):
                    sg = sv[l]
                    t = j * 16 + l
                    rowv = tuple(rows2[p][t, pl.ds(q * 16, 16)]
                                 for q in range(NQ))

                    def flush(ops):
                        cur_ = ops[0]
                        for q in range(NQ):
                            sl = pl.ds(q * 16, 16)
                            acc[cur_, sl] = acc[cur_, sl] + ops[1 + q]
                        return (sg,) + ops[1 + NQ:]

                    def cont(ops):
                        return ((ops[0],)
                                + tuple(a + r for a, r in
                                        zip(ops[1:1 + NQ], ops[1 + NQ:])))

                    st2 = lax.cond(sg != st2[0], flush, cont, st2 + rowv)
                return st2

            return lax.fori_loop(0, CH // 16, inner, st)

        st = lax.cond(par == 0,
                      lambda s: rmw(0, s),
                      lambda s: rmw(1, s),
                      carry[1:])
        return (cr,) + st

    fin = lax.fori_loop(0, nch, chunk, init)
    cur = fin[1]
    for q in range(NQ):
        sl = pl.ds(q * 16, 16)
        acc[cur, sl] = acc[cur, sl] + fin[2 + q]

    pltpu.sync_copy(acc.at[pl.ds(0, BAGS_PER_W)],
                    x_hbm.at[pl.ds(bag0, BAGS_PER_W)])


_sc_embed_call = functools.partial(
    pl.kernel,
    out_type=jax.ShapeDtypeStruct((N_BAGS, H), jnp.float32),
    mesh=plsc.VectorSubcoreMesh(core_axis_name="c", subcore_axis_name="s"),
    compiler_params=pltpu.CompilerParams(needs_layout_passes=False),
    scratch_types=[
        pltpu.VMEM((CH,), jnp.int32),
        pltpu.VMEM((CH,), jnp.int32),
        pltpu.VMEM((32,), jnp.int32),
        pltpu.VMEM((CH, H), jnp.float32),
        pltpu.VMEM((CH, H), jnp.float32),
        pltpu.VMEM((400,), jnp.int32),
        pltpu.VMEM((CH,), jnp.int32),
        pltpu.VMEM((BAGS_PER_W + 1, H), jnp.float32),
        pltpu.SemaphoreType.DMA,
        pltpu.SemaphoreType.DMA,
    ],
)(_sc_embed)


def _tc_classifier(x_ref, w_ref, b_ref, o_ref):
    y = jnp.dot(x_ref[...], w_ref[...], preferred_element_type=jnp.float32)
    y = jnp.maximum(y + b_ref[...], 0.0)
    y = y - jnp.max(y, axis=1, keepdims=True)
    o_ref[...] = jnp.exp(y)


def kernel(char_ids, offsets, prev_tag_ids, word_ids,
           char_table, tag_table, word_table, W_w, W_b):
    cid_pad = jnp.concatenate(
        [char_ids.astype(jnp.int32), jnp.zeros((CH,), jnp.int32)])
    ctab_rep = jnp.tile(char_table, (8, 1))
    off_pad = jnp.concatenate(
        [offsets.astype(jnp.int32), jnp.full((16,), TOTAL, jnp.int32)])
    wf = word_ids.reshape(-1).astype(jnp.int32)

    x = _sc_embed_call(cid_pad, off_pad, wf,
                       prev_tag_ids.astype(jnp.int32),
                       ctab_rep, tag_table, word_table)
    x = x.reshape(B, 3 * H)

    blk = 512
    out = pl.pallas_call(
        _tc_classifier,
        grid=(B // blk,),
        in_specs=[
            pl.BlockSpec((blk, 3 * H), lambda i: (i, 0)),
            pl.BlockSpec((3 * H, TAGS), lambda i: (0, 0)),
            pl.BlockSpec((1, TAGS), lambda i: (0, 0)),
        ],
        out_specs=pl.BlockSpec((blk, TAGS), lambda i: (i, 0)),
        out_shape=jax.ShapeDtypeStruct((B, TAGS), jnp.float32),
    )(x, W_w.T, W_b.reshape(1, TAGS))
    return out
